# baseline (speedup 1.0000x reference)
import functools

import jax
import jax.numpy as jnp
from jax import lax
from jax.experimental import pallas as pl
from jax.experimental.pallas import tpu as pltpu
from jax.experimental.pallas import tpu_sc as plsc

_DIM = 128
_BATCH = 16384
_CHUNK = 128


@functools.lru_cache(maxsize=None)
def _build_gather():
    info = plsc.get_sparse_core_info()
    nw = info.num_cores * info.num_subcores
    bpw = _BATCH // nw
    nchunks = bpw // _CHUNK
    mesh = plsc.VectorSubcoreMesh(core_axis_name="c", subcore_axis_name="s")

    @functools.partial(
        pl.kernel,
        mesh=mesh,
        out_type=jax.ShapeDtypeStruct((_BATCH, _DIM), jnp.float32),
        scratch_types=[
            pltpu.VMEM((bpw,), jnp.int32),
            pltpu.VMEM((_CHUNK, _DIM), jnp.float32),
            pltpu.VMEM((_CHUNK, _DIM), jnp.float32),
            pltpu.SemaphoreType.DMA,
            pltpu.SemaphoreType.DMA,
        ],
    )
    def gather(idx_hbm, table_hbm, out_hbm, idx_v, rows0, rows1, gsem, ssem):
        wid = lax.axis_index("s") * info.num_cores + lax.axis_index("c")
        base = wid * bpw
        pltpu.sync_copy(idx_hbm.at[pl.ds(base, bpw)], idx_v)
        bufs = (rows0, rows1)

        def g(j):
            return pltpu.async_copy(
                table_hbm.at[idx_v.at[pl.ds(j * _CHUNK, _CHUNK)]],
                bufs[j % 2], gsem)

        def s(j):
            return pltpu.async_copy(
                bufs[j % 2],
                out_hbm.at[pl.ds(base + j * _CHUNK, _CHUNK)], ssem)

        gh = [None] * nchunks
        sh = [None] * nchunks
        gh[0] = g(0)
        for j in range(nchunks):
            gh[j].wait()
            if j + 1 < nchunks:
                if j >= 1:
                    sh[j - 1].wait()
                gh[j + 1] = g(j + 1)
            sh[j] = s(j)
        sh[nchunks - 2].wait()
        sh[nchunks - 1].wait()

    return gather


@jax.jit
def kernel(timesteps, pe):
    return _build_gather()(timesteps.astype(jnp.int32), pe)
